# Initial kernel scaffold; baseline (speedup 1.0000x reference)
import jax
import jax.numpy as jnp
from jax.experimental import pallas as pl


def kernel(node_emb, pos_edges, neg_edges):
    raise NotImplementedError("write your pallas kernel here")


# speedup vs baseline: 5.7107x; 5.7107x over previous
import functools

import jax
import jax.numpy as jnp
from jax import lax
from jax.experimental import pallas as pl
from jax.experimental.pallas import tpu as pltpu
from jax.experimental.pallas import tpu_sc as plsc

N_NODES = 10000
D_FEAT = 128
N_EDGES = 320000
MARGIN = 50.0

NC = 2
NS = 16
NW = NC * NS
EPW = N_EDGES // NW
CH = 80
NCH = EPW // CH

_mesh = plsc.VectorSubcoreMesh(core_axis_name="c", subcore_axis_name="s")


@functools.partial(
    pl.kernel,
    out_type=jax.ShapeDtypeStruct((NW, 16), jnp.float32),
    mesh=_mesh,
    compiler_params=pltpu.CompilerParams(needs_layout_passes=False),
    scratch_types=[
        pltpu.VMEM((NCH, CH), jnp.int32),
        pltpu.VMEM((NCH, CH), jnp.int32),
        pltpu.VMEM((NCH, CH), jnp.int32),
        pltpu.VMEM((NCH, CH), jnp.int32),
        pltpu.VMEM((CH, D_FEAT), jnp.float32),
        pltpu.VMEM((CH, D_FEAT), jnp.float32),
        pltpu.VMEM((CH, D_FEAT), jnp.float32),
        pltpu.VMEM((CH, D_FEAT), jnp.float32),
        pltpu.VMEM((CH, 16), jnp.float32),
        pltpu.VMEM((1, 16), jnp.float32),
        pltpu.SemaphoreType.DMA,
    ],
)
def _edge_loss_partials(tbl, pa, pb, na, nb, out,
                        idx_pa, idx_pb, idx_na, idx_nb,
                        rows_pa, rows_pb, rows_na, rows_nb,
                        dvec, accv, sem):
    c = lax.axis_index("c")
    s = lax.axis_index("s")
    wid = s * NC + c

    pltpu.sync_copy(pa.at[wid], idx_pa)
    pltpu.sync_copy(pb.at[wid], idx_pb)
    pltpu.sync_copy(na.at[wid], idx_na)
    pltpu.sync_copy(nb.at[wid], idx_nb)

    lane = lax.iota(jnp.int32, 16)

    def chunk_body(i, acc):
        cp1 = pltpu.async_copy(tbl.at[idx_pa.at[i]], rows_pa, sem)
        cp2 = pltpu.async_copy(tbl.at[idx_pb.at[i]], rows_pb, sem)
        cp3 = pltpu.async_copy(tbl.at[idx_na.at[i]], rows_na, sem)
        cp4 = pltpu.async_copy(tbl.at[idx_nb.at[i]], rows_nb, sem)
        cp1.wait()
        cp2.wait()
        cp3.wait()
        cp4.wait()

        def edge_body(e, carry):
            sl = pl.ds(0, 16)
            d = rows_na[e, sl] * rows_nb[e, sl] - rows_pa[e, sl] * rows_pb[e, sl]
            for g in range(1, D_FEAT // 16):
                sl = pl.ds(16 * g, 16)
                d = d + rows_na[e, sl] * rows_nb[e, sl]
                d = d - rows_pa[e, sl] * rows_pb[e, sl]
            dvec[e, :] = d
            return carry

        lax.fori_loop(0, CH, edge_body, 0)

        def block_body(b, acc):
            rows16 = b * 16 + lane
            t = plsc.load_gather(dvec, [rows16, jnp.zeros((16,), jnp.int32)])
            for l in range(1, 16):
                t = t + plsc.load_gather(
                    dvec, [rows16, jnp.full((16,), l, jnp.int32)])
            return acc + jnp.maximum(t + MARGIN, jnp.float32(0.0))

        return lax.fori_loop(0, CH // 16, block_body, acc)

    acc = lax.fori_loop(0, NCH, chunk_body, jnp.zeros((16,), jnp.float32))

    accv[0, :] = acc
    pltpu.sync_copy(accv, out.at[pl.ds(wid, 1)])


def _sum_body(p_ref, o_ref):
    o_ref[...] = jnp.sum(p_ref[...], keepdims=True)


def kernel(node_emb, pos_edges, neg_edges):
    pos_edges = pos_edges.astype(jnp.int32)
    neg_edges = neg_edges.astype(jnp.int32)
    pa = pos_edges[:, 0].reshape(NW, NCH, CH)
    pb = pos_edges[:, 1].reshape(NW, NCH, CH)
    na = neg_edges[:, 0].reshape(NW, NCH, CH)
    nb = neg_edges[:, 1].reshape(NW, NCH, CH)

    partials = _edge_loss_partials(node_emb, pa, pb, na, nb)

    loss = pl.pallas_call(
        _sum_body,
        out_shape=jax.ShapeDtypeStruct((1, 1), jnp.float32),
    )(partials)
    return loss[0, 0]
